# baseline (speedup 1.0000x reference)
import functools

import jax
import jax.numpy as jnp
from jax import lax
from jax.experimental import pallas as pl
from jax.experimental.pallas import tpu as pltpu
from jax.experimental.pallas import tpu_sc as plsc

B, N, S, K = 8, 4096, 1024, 64
C = 128
EPS = 1e-5
R = B * S * K
TILE = 4096
GRID = R // TILE
SG = TILE // K
NW = 32
RPW = R // NW
CH = 128
CPW = (B * S) // NW

_HI = jax.lax.Precision.HIGHEST


def _bn_affine(stats, gamma, beta, m):
    s, q = stats[0], stats[1]
    mean = s / m
    var = q / m - mean * mean
    scale = gamma / jnp.sqrt(var + EPS)
    shift = beta - mean * scale
    return scale.reshape(1, -1), shift.reshape(1, -1)


def _pack(conv_f32, geo_f32):
    lo = lax.bitcast_convert_type(conv_f32.astype(jnp.bfloat16),
                                  jnp.uint16).astype(jnp.uint32)
    hi = lax.bitcast_convert_type(geo_f32.astype(jnp.bfloat16),
                                  jnp.uint16).astype(jnp.uint32)
    return lax.bitcast_convert_type(lo | (hi << 16), jnp.int32)


def _unpack(words_i32):
    w = lax.bitcast_convert_type(words_i32, jnp.uint32)
    conv = lax.bitcast_convert_type(w << 16, jnp.float32)
    geo = lax.bitcast_convert_type(w & jnp.uint32(0xFFFF0000), jnp.float32)
    return conv, geo


def _tables_body(f_ref, p_ref, wf_ref, m16_ref, mc16_ref, t_ref, ct_ref):
    t = jnp.dot(p_ref[...], m16_ref[...],
                preferred_element_type=jnp.float32, precision=_HI)
    tl = t[:, :C] + jnp.dot(f_ref[...].astype(jnp.bfloat16), wf_ref[...],
                            preferred_element_type=jnp.float32)
    t_ref[...] = _pack(tl, t[:, C:])
    ct = jnp.dot(p_ref[...], mc16_ref[...],
                 preferred_element_type=jnp.float32, precision=_HI)
    ct_ref[...] = _pack(ct[:, :C], ct[:, C:])


def _tables(feat2d, p16, w1f_t, m16, mc16):
    return pl.pallas_call(
        _tables_body,
        grid=(B * N // 512,),
        in_specs=[pl.BlockSpec((512, C), lambda i: (i, 0)),
                  pl.BlockSpec((512, 16), lambda i: (i, 0)),
                  pl.BlockSpec((C, C), lambda i: (0, 0)),
                  pl.BlockSpec((16, 2 * C), lambda i: (0, 0)),
                  pl.BlockSpec((16, 2 * C), lambda i: (0, 0))],
        out_specs=[pl.BlockSpec((512, C), lambda i: (i, 0)),
                   pl.BlockSpec((512, C), lambda i: (i, 0))],
        out_shape=[jax.ShapeDtypeStruct((B * N, C), jnp.int32),
                   jax.ShapeDtypeStruct((B * N, C), jnp.int32)],
        compiler_params=pltpu.CompilerParams(
            dimension_semantics=("parallel",)),
    )(feat2d, p16, w1f_t, m16, mc16)


def _sc_gather(t, ct, nidx, cent):
    mesh = plsc.VectorSubcoreMesh(core_axis_name="c", subcore_axis_name="s")

    @functools.partial(
        pl.kernel, mesh=mesh,
        out_type=[jax.ShapeDtypeStruct((R, C), jnp.int32),
                  jax.ShapeDtypeStruct((B * S, C), jnp.int32)],
        scratch_types=[pltpu.VMEM((CH,), jnp.int32),
                       pltpu.VMEM((CH, C), jnp.int32),
                       pltpu.SemaphoreType.DMA],
    )
    def k(t_hbm, ct_hbm, nidx_hbm, cent_hbm, g_hbm, cpt_hbm,
          idx_v, rows_v, sem):
        wid = lax.axis_index("s") * 2 + lax.axis_index("c")
        b_off = (wid // 4) * N

        for cb in range(CPW // CH):
            cbase = wid * CPW + cb * CH
            pltpu.sync_copy(cent_hbm.at[pl.ds(cbase, CH)], idx_v)
            for j in range(CH // 16):
                idx_v[pl.ds(j * 16, 16)] = idx_v[pl.ds(j * 16, 16)] + b_off
            pltpu.async_copy(ct_hbm.at[idx_v], rows_v, sem).wait()
            pltpu.sync_copy(rows_v, cpt_hbm.at[pl.ds(cbase, CH)])

        def body(i, carry):
            base = wid * RPW + i * CH
            pltpu.sync_copy(nidx_hbm.at[pl.ds(base, CH)], idx_v)
            for j in range(CH // 16):
                idx_v[pl.ds(j * 16, 16)] = idx_v[pl.ds(j * 16, 16)] + b_off
            pltpu.async_copy(t_hbm.at[idx_v], rows_v, sem).wait()
            pltpu.sync_copy(rows_v, g_hbm.at[pl.ds(base, CH)])
            return carry
        lax.fori_loop(0, RPW // CH, body, 0)

    return k(t, ct, nidx, cent)


def _l1_body(g_ref, cpt_ref, bias_ref, st_ref):
    cpc, cpg = _unpack(cpt_ref[...])
    gc, gg = _unpack(g_ref[...])
    ccc = cpc + bias_ref[:, :C]
    ccg = cpg + bias_ref[:, C:]
    hc = (gc.reshape(SG, K, C) + ccc[:, None, :]).reshape(TILE, C)
    hg = (gg.reshape(SG, K, C) + ccg[:, None, :]).reshape(TILE, C)
    ones = jnp.ones((1, TILE), jnp.bfloat16)
    hcb = hc.astype(jnp.bfloat16)
    hgb = hg.astype(jnp.bfloat16)
    s = jnp.concatenate(
        [jnp.dot(ones, hcb, preferred_element_type=jnp.float32),
         jnp.dot(ones, hgb, preferred_element_type=jnp.float32)], axis=1)
    q = jnp.concatenate(
        [jnp.dot(ones, hcb * hcb, preferred_element_type=jnp.float32),
         jnp.dot(ones, hgb * hgb, preferred_element_type=jnp.float32)],
        axis=1)
    st_ref[...] = jnp.concatenate(
        [s, q, jnp.zeros((6, 2 * C), jnp.float32)], axis=0)[None]


def _l1(g, cpt, bias):
    return pl.pallas_call(
        _l1_body,
        grid=(GRID,),
        in_specs=[pl.BlockSpec((TILE, C), lambda i: (i, 0)),
                  pl.BlockSpec((SG, C), lambda i: (i, 0)),
                  pl.BlockSpec((1, 2 * C), lambda i: (0, 0))],
        out_specs=pl.BlockSpec((1, 8, 2 * C), lambda i: (i, 0, 0)),
        out_shape=jax.ShapeDtypeStruct((GRID, 8, 2 * C), jnp.float32),
        compiler_params=pltpu.CompilerParams(
            dimension_semantics=("parallel",)),
    )(g, cpt, bias)


def _assemble_x1(g_ref, cpt_ref, bias_ref, sc_ref, sh_ref):
    cpc, cpg = _unpack(cpt_ref[...])
    gc, gg = _unpack(g_ref[...])
    ccc = cpc + bias_ref[:, :C]
    ccg = cpg + bias_ref[:, C:]
    hc = (gc.reshape(SG, K, C) + ccc[:, None, :]).reshape(TILE, C)
    hg = (gg.reshape(SG, K, C) + ccg[:, None, :]).reshape(TILE, C)
    xc = jnp.maximum(hc * sc_ref[:, :C] + sh_ref[:, :C], 0.0)
    xg = jnp.maximum(hg * sc_ref[:, C:] + sh_ref[:, C:], 0.0)
    return xc, xg


def _mid_body(g_ref, cpt_ref, bias_ref, sc_ref, sh_ref, wc_ref, wg_ref,
              b_ref, oc_ref, og_ref, st_ref):
    xc, xg = _assemble_x1(g_ref, cpt_ref, bias_ref, sc_ref, sh_ref)
    yc = jnp.dot(xc.astype(jnp.bfloat16), wc_ref[...],
                 preferred_element_type=jnp.float32) + b_ref[:, :C]
    yg = jnp.dot(xg.astype(jnp.bfloat16), wg_ref[...],
                 preferred_element_type=jnp.float32) + b_ref[:, C:]
    ycb = yc.astype(jnp.bfloat16)
    ygb = yg.astype(jnp.bfloat16)
    oc_ref[...] = ycb
    og_ref[...] = ygb
    ones = jnp.ones((1, TILE), jnp.bfloat16)
    s = jnp.concatenate(
        [jnp.dot(ones, ycb, preferred_element_type=jnp.float32),
         jnp.dot(ones, ygb, preferred_element_type=jnp.float32)], axis=1)
    q = jnp.concatenate(
        [jnp.dot(ones, ycb * ycb, preferred_element_type=jnp.float32),
         jnp.dot(ones, ygb * ygb, preferred_element_type=jnp.float32)],
        axis=1)
    st_ref[...] = jnp.concatenate(
        [s, q, jnp.zeros((6, 2 * C), jnp.float32)], axis=0)[None]


def _mid(g, cpt, bias, scale, shift, wc, wg, b):
    return pl.pallas_call(
        _mid_body,
        grid=(GRID,),
        in_specs=[pl.BlockSpec((TILE, C), lambda i: (i, 0)),
                  pl.BlockSpec((SG, C), lambda i: (i, 0)),
                  pl.BlockSpec((1, 2 * C), lambda i: (0, 0)),
                  pl.BlockSpec((1, 2 * C), lambda i: (0, 0)),
                  pl.BlockSpec((1, 2 * C), lambda i: (0, 0)),
                  pl.BlockSpec((C, C), lambda i: (0, 0)),
                  pl.BlockSpec((C, C), lambda i: (0, 0)),
                  pl.BlockSpec((1, 2 * C), lambda i: (0, 0))],
        out_specs=[pl.BlockSpec((TILE, C), lambda i: (i, 0)),
                   pl.BlockSpec((TILE, C), lambda i: (i, 0)),
                   pl.BlockSpec((1, 8, 2 * C), lambda i: (i, 0, 0))],
        out_shape=[jax.ShapeDtypeStruct((R, C), jnp.bfloat16),
                   jax.ShapeDtypeStruct((R, C), jnp.bfloat16),
                   jax.ShapeDtypeStruct((GRID, 8, 2 * C), jnp.float32)],
        compiler_params=pltpu.CompilerParams(
            dimension_semantics=("parallel",)),
    )(g, cpt, bias, scale, shift, wc, wg, b)


def _l3_body(hc_ref, hg_ref, sc_ref, sh_ref, wc_ref, wg_ref, b_ref,
             mx_ref, mn_ref, st_ref):
    xc = jnp.maximum(
        hc_ref[...].astype(jnp.float32) * sc_ref[:, :C] + sh_ref[:, :C], 0.0)
    xg = jnp.maximum(
        hg_ref[...].astype(jnp.float32) * sc_ref[:, C:] + sh_ref[:, C:], 0.0)
    yc = jnp.dot(xc.astype(jnp.bfloat16), wc_ref[...],
                 preferred_element_type=jnp.float32) + b_ref[:, :2 * C]
    yg = jnp.dot(xg.astype(jnp.bfloat16), wg_ref[...],
                 preferred_element_type=jnp.float32) + b_ref[:, 2 * C:]
    yc3 = yc.reshape(SG, K, 2 * C)
    yg3 = yg.reshape(SG, K, 2 * C)
    mx_ref[...] = jnp.concatenate([jnp.max(yc3, axis=1),
                                   jnp.max(yg3, axis=1)], axis=1)
    mn_ref[...] = jnp.concatenate([jnp.min(yc3, axis=1),
                                   jnp.min(yg3, axis=1)], axis=1)
    ycb = yc.astype(jnp.bfloat16)
    ygb = yg.astype(jnp.bfloat16)
    ones = jnp.ones((1, TILE), jnp.bfloat16)
    s = jnp.concatenate(
        [jnp.dot(ones, ycb, preferred_element_type=jnp.float32),
         jnp.dot(ones, ygb, preferred_element_type=jnp.float32)], axis=1)
    q = jnp.concatenate(
        [jnp.dot(ones, ycb * ycb, preferred_element_type=jnp.float32),
         jnp.dot(ones, ygb * ygb, preferred_element_type=jnp.float32)],
        axis=1)
    st_ref[...] = jnp.concatenate(
        [s, q, jnp.zeros((6, 4 * C), jnp.float32)], axis=0)[None]


def _l3(hc, hg, scale, shift, wc, wg, b):
    return pl.pallas_call(
        _l3_body,
        grid=(GRID,),
        in_specs=[pl.BlockSpec((TILE, C), lambda i: (i, 0)),
                  pl.BlockSpec((TILE, C), lambda i: (i, 0)),
                  pl.BlockSpec((1, 2 * C), lambda i: (0, 0)),
                  pl.BlockSpec((1, 2 * C), lambda i: (0, 0)),
                  pl.BlockSpec((C, 2 * C), lambda i: (0, 0)),
                  pl.BlockSpec((C, 2 * C), lambda i: (0, 0)),
                  pl.BlockSpec((1, 4 * C), lambda i: (0, 0))],
        out_specs=[pl.BlockSpec((SG, 4 * C), lambda i: (i, 0)),
                   pl.BlockSpec((SG, 4 * C), lambda i: (i, 0)),
                   pl.BlockSpec((1, 8, 4 * C), lambda i: (i, 0, 0))],
        out_shape=[jax.ShapeDtypeStruct((B * S, 4 * C), jnp.float32),
                   jax.ShapeDtypeStruct((B * S, 4 * C), jnp.float32),
                   jax.ShapeDtypeStruct((GRID, 8, 4 * C), jnp.float32)],
        compiler_params=pltpu.CompilerParams(
            dimension_semantics=("parallel",)),
    )(hc, hg, scale, shift, wc, wg, b)


def _fin_body(mx_ref, mn_ref, a_ref, c_ref, o_ref):
    a = a_ref[...]
    sel = jnp.where(a >= 0.0, mx_ref[...], mn_ref[...])
    v = jnp.maximum(sel * a + c_ref[...], 0.0)
    o_ref[0] = v[:, :2 * C]
    o_ref[1] = v[:, 2 * C:]


def _fin(mx, mn, a, c):
    return pl.pallas_call(
        _fin_body,
        grid=(B * S // TILE,),
        in_specs=[pl.BlockSpec((TILE, 4 * C), lambda i: (i, 0)),
                  pl.BlockSpec((TILE, 4 * C), lambda i: (i, 0)),
                  pl.BlockSpec((1, 4 * C), lambda i: (0, 0)),
                  pl.BlockSpec((1, 4 * C), lambda i: (0, 0))],
        out_specs=pl.BlockSpec((2, TILE, 2 * C), lambda i: (0, i, 0)),
        out_shape=jax.ShapeDtypeStruct((2, B * S, 2 * C), jnp.float32),
        compiler_params=pltpu.CompilerParams(
            dimension_semantics=("parallel",)),
    )(mx, mn, a, c)


def kernel(pos, feat, centroids, neighbor_idx, params):
    conv = params["conv"]
    geo = params["conv_geo"]

    w1 = conv[0]["W"]
    w1r_t = jnp.transpose(w1[:, :3])
    w1f_t = jnp.transpose(w1[:, 3:])
    g1 = geo[0]["W"]
    gsrc_t = jnp.transpose(g1[:, :3])
    gdst_t = jnp.transpose(g1[:, 3:])

    m16 = jnp.zeros((16, 2 * C), jnp.float32)
    m16 = m16.at[0:3, 0:C].set(w1r_t).at[0:3, C:].set(gsrc_t)
    mc16 = jnp.zeros((16, 2 * C), jnp.float32)
    mc16 = mc16.at[0:3, 0:C].set(-w1r_t).at[0:3, C:].set(gdst_t)
    bias1 = jnp.concatenate([conv[0]["b"], geo[0]["b"]]).reshape(1, 2 * C)

    w2c = conv[1]["W"].T.astype(jnp.bfloat16)
    w2g = geo[1]["W"].T.astype(jnp.bfloat16)
    bias2 = jnp.concatenate([conv[1]["b"], geo[1]["b"]]).reshape(1, 2 * C)

    w3c = conv[2]["W"].T.astype(jnp.bfloat16)
    w3g = geo[2]["W"].T.astype(jnp.bfloat16)
    bias3 = jnp.concatenate([conv[2]["b"], geo[2]["b"]]).reshape(1, 4 * C)

    gam1 = jnp.concatenate([conv[0]["gamma"], geo[0]["gamma"]])
    bet1 = jnp.concatenate([conv[0]["beta"], geo[0]["beta"]])
    gam2 = jnp.concatenate([conv[1]["gamma"], geo[1]["gamma"]])
    bet2 = jnp.concatenate([conv[1]["beta"], geo[1]["beta"]])
    gam3 = jnp.concatenate([conv[2]["gamma"], geo[2]["gamma"]])
    bet3 = jnp.concatenate([conv[2]["beta"], geo[2]["beta"]])

    feat2d = feat.reshape(B * N, C)
    p16 = jnp.zeros((B * N, 16), jnp.float32).at[:, :3].set(pos.reshape(-1, 3))
    nidx = neighbor_idx.reshape(-1).astype(jnp.int32)
    cent = centroids.reshape(-1).astype(jnp.int32)

    t, ct = _tables(feat2d, p16, w1f_t.astype(jnp.bfloat16), m16, mc16)
    g, cpt = _sc_gather(t, ct, nidx, cent)
    st1 = jnp.sum(_l1(g, cpt, bias1), axis=0)
    m = float(R)
    sc1, sh1 = _bn_affine(st1, gam1, bet1, m)
    h2c, h2g, st2p = _mid(g, cpt, bias1, sc1, sh1, w2c, w2g, bias2)
    sc2, sh2 = _bn_affine(jnp.sum(st2p, axis=0), gam2, bet2, m)
    mx, mn, st3p = _l3(h2c, h2g, sc2, sh2, w3c, w3g, bias3)
    st3 = jnp.sum(st3p, axis=0)
    sc3, sh3 = _bn_affine(st3, gam3, bet3, m)
    out = _fin(mx, mn, sc3, sh3)

    return out.reshape(2 * B * S, 2 * C)


# speedup vs baseline: 1.0955x; 1.0955x over previous
import functools

import jax
import jax.numpy as jnp
from jax import lax
from jax.experimental import pallas as pl
from jax.experimental.pallas import tpu as pltpu
from jax.experimental.pallas import tpu_sc as plsc

B, N, S, K = 8, 4096, 1024, 64
C = 128
EPS = 1e-5
R = B * S * K
TILE = 8192
TILE5 = 4096
GRID = R // TILE
SG = TILE // K
NW = 32
RPW = R // NW
CH = 128
CPW = (B * S) // NW

_HI = jax.lax.Precision.HIGHEST


def _bn_affine(stats, gamma, beta, m):
    s, q = stats[0], stats[1]
    mean = s / m
    var = q / m - mean * mean
    scale = gamma / jnp.sqrt(var + EPS)
    shift = beta - mean * scale
    return scale.reshape(1, -1), shift.reshape(1, -1)


def _pack(conv_f32, geo_f32):
    lo = lax.bitcast_convert_type(conv_f32.astype(jnp.bfloat16),
                                  jnp.uint16).astype(jnp.uint32)
    hi = lax.bitcast_convert_type(geo_f32.astype(jnp.bfloat16),
                                  jnp.uint16).astype(jnp.uint32)
    return lax.bitcast_convert_type(lo | (hi << 16), jnp.int32)


def _unpack(words_i32):
    w = lax.bitcast_convert_type(words_i32, jnp.uint32)
    conv = lax.bitcast_convert_type(w << 16, jnp.float32)
    geo = lax.bitcast_convert_type(w & jnp.uint32(0xFFFF0000), jnp.float32)
    return conv, geo


def _tables_body(f_ref, p_ref, wf_ref, m16_ref, mc16_ref, t_ref, ct_ref):
    t = jnp.dot(p_ref[...], m16_ref[...],
                preferred_element_type=jnp.float32, precision=_HI)
    tl = t[:, :C] + jnp.dot(f_ref[...].astype(jnp.bfloat16), wf_ref[...],
                            preferred_element_type=jnp.float32)
    t_ref[...] = _pack(tl, t[:, C:])
    ct = jnp.dot(p_ref[...], mc16_ref[...],
                 preferred_element_type=jnp.float32, precision=_HI)
    ct_ref[...] = _pack(ct[:, :C], ct[:, C:])


def _tables(feat2d, p16, w1f_t, m16, mc16):
    return pl.pallas_call(
        _tables_body,
        grid=(B * N // 512,),
        in_specs=[pl.BlockSpec((512, C), lambda i: (i, 0)),
                  pl.BlockSpec((512, 16), lambda i: (i, 0)),
                  pl.BlockSpec((C, C), lambda i: (0, 0)),
                  pl.BlockSpec((16, 2 * C), lambda i: (0, 0)),
                  pl.BlockSpec((16, 2 * C), lambda i: (0, 0))],
        out_specs=[pl.BlockSpec((512, C), lambda i: (i, 0)),
                   pl.BlockSpec((512, C), lambda i: (i, 0))],
        out_shape=[jax.ShapeDtypeStruct((B * N, C), jnp.int32),
                   jax.ShapeDtypeStruct((B * N, C), jnp.int32)],
        compiler_params=pltpu.CompilerParams(
            dimension_semantics=("parallel",)),
    )(feat2d, p16, w1f_t, m16, mc16)


def _sc_gather(t, ct, nidx, cent):
    mesh = plsc.VectorSubcoreMesh(core_axis_name="c", subcore_axis_name="s")

    @functools.partial(
        pl.kernel, mesh=mesh,
        out_type=[jax.ShapeDtypeStruct((R, C), jnp.int32),
                  jax.ShapeDtypeStruct((B * S, C), jnp.int32)],
        scratch_types=[pltpu.VMEM((CH,), jnp.int32),
                       pltpu.VMEM((CH, C), jnp.int32),
                       pltpu.SemaphoreType.DMA],
    )
    def k(t_hbm, ct_hbm, nidx_hbm, cent_hbm, g_hbm, cpt_hbm,
          idx_v, rows_v, sem):
        wid = lax.axis_index("s") * 2 + lax.axis_index("c")
        b_off = (wid // 4) * N

        for cb in range(CPW // CH):
            cbase = wid * CPW + cb * CH
            pltpu.sync_copy(cent_hbm.at[pl.ds(cbase, CH)], idx_v)
            for j in range(CH // 16):
                idx_v[pl.ds(j * 16, 16)] = idx_v[pl.ds(j * 16, 16)] + b_off
            pltpu.async_copy(ct_hbm.at[idx_v], rows_v, sem).wait()
            pltpu.sync_copy(rows_v, cpt_hbm.at[pl.ds(cbase, CH)])

        def body(i, carry):
            base = wid * RPW + i * CH
            pltpu.sync_copy(nidx_hbm.at[pl.ds(base, CH)], idx_v)
            for j in range(CH // 16):
                idx_v[pl.ds(j * 16, 16)] = idx_v[pl.ds(j * 16, 16)] + b_off
            pltpu.async_copy(t_hbm.at[idx_v], rows_v, sem).wait()
            pltpu.sync_copy(rows_v, g_hbm.at[pl.ds(base, CH)])
            return carry
        lax.fori_loop(0, RPW // CH, body, 0)

    return k(t, ct, nidx, cent)


def _l1_body(g_ref, cpt_ref, bias_ref, st_ref):
    cpc, cpg = _unpack(cpt_ref[...])
    gc, gg = _unpack(g_ref[...])
    ccc = cpc + bias_ref[:, :C]
    ccg = cpg + bias_ref[:, C:]
    hc = (gc.reshape(SG, K, C) + ccc[:, None, :]).reshape(TILE, C)
    hg = (gg.reshape(SG, K, C) + ccg[:, None, :]).reshape(TILE, C)
    s = jnp.concatenate([jnp.sum(hc, axis=0, keepdims=True),
                         jnp.sum(hg, axis=0, keepdims=True)], axis=1)
    q = jnp.concatenate([jnp.sum(hc * hc, axis=0, keepdims=True),
                         jnp.sum(hg * hg, axis=0, keepdims=True)], axis=1)
    st_ref[...] = jnp.concatenate(
        [s, q, jnp.zeros((6, 2 * C), jnp.float32)], axis=0)[None]


def _l1(g, cpt, bias):
    return pl.pallas_call(
        _l1_body,
        grid=(GRID,),
        in_specs=[pl.BlockSpec((TILE, C), lambda i: (i, 0)),
                  pl.BlockSpec((SG, C), lambda i: (i, 0)),
                  pl.BlockSpec((1, 2 * C), lambda i: (0, 0))],
        out_specs=pl.BlockSpec((1, 8, 2 * C), lambda i: (i, 0, 0)),
        out_shape=jax.ShapeDtypeStruct((GRID, 8, 2 * C), jnp.float32),
        compiler_params=pltpu.CompilerParams(
            dimension_semantics=("parallel",)),
    )(g, cpt, bias)


def _assemble_x1(g_ref, cpt_ref, bias_ref, sc_ref, sh_ref):
    cpc, cpg = _unpack(cpt_ref[...])
    gc, gg = _unpack(g_ref[...])
    ccc = cpc + bias_ref[:, :C]
    ccg = cpg + bias_ref[:, C:]
    hc = (gc.reshape(SG, K, C) + ccc[:, None, :]).reshape(TILE, C)
    hg = (gg.reshape(SG, K, C) + ccg[:, None, :]).reshape(TILE, C)
    xc = jnp.maximum(hc * sc_ref[:, :C] + sh_ref[:, :C], 0.0)
    xg = jnp.maximum(hg * sc_ref[:, C:] + sh_ref[:, C:], 0.0)
    return xc, xg


def _mid_body(g_ref, cpt_ref, bias_ref, sc_ref, sh_ref, wc_ref, wg_ref,
              b_ref, oc_ref, og_ref, st_ref):
    xc, xg = _assemble_x1(g_ref, cpt_ref, bias_ref, sc_ref, sh_ref)
    yc = jnp.dot(xc.astype(jnp.bfloat16), wc_ref[...],
                 preferred_element_type=jnp.float32) + b_ref[:, :C]
    yg = jnp.dot(xg.astype(jnp.bfloat16), wg_ref[...],
                 preferred_element_type=jnp.float32) + b_ref[:, C:]
    oc_ref[...] = yc.astype(jnp.bfloat16)
    og_ref[...] = yg.astype(jnp.bfloat16)
    s = jnp.concatenate([jnp.sum(yc, axis=0, keepdims=True),
                         jnp.sum(yg, axis=0, keepdims=True)], axis=1)
    q = jnp.concatenate([jnp.sum(yc * yc, axis=0, keepdims=True),
                         jnp.sum(yg * yg, axis=0, keepdims=True)], axis=1)
    st_ref[...] = jnp.concatenate(
        [s, q, jnp.zeros((6, 2 * C), jnp.float32)], axis=0)[None]


def _mid(g, cpt, bias, scale, shift, wc, wg, b):
    return pl.pallas_call(
        _mid_body,
        grid=(GRID,),
        in_specs=[pl.BlockSpec((TILE, C), lambda i: (i, 0)),
                  pl.BlockSpec((SG, C), lambda i: (i, 0)),
                  pl.BlockSpec((1, 2 * C), lambda i: (0, 0)),
                  pl.BlockSpec((1, 2 * C), lambda i: (0, 0)),
                  pl.BlockSpec((1, 2 * C), lambda i: (0, 0)),
                  pl.BlockSpec((C, C), lambda i: (0, 0)),
                  pl.BlockSpec((C, C), lambda i: (0, 0)),
                  pl.BlockSpec((1, 2 * C), lambda i: (0, 0))],
        out_specs=[pl.BlockSpec((TILE, C), lambda i: (i, 0)),
                   pl.BlockSpec((TILE, C), lambda i: (i, 0)),
                   pl.BlockSpec((1, 8, 2 * C), lambda i: (i, 0, 0))],
        out_shape=[jax.ShapeDtypeStruct((R, C), jnp.bfloat16),
                   jax.ShapeDtypeStruct((R, C), jnp.bfloat16),
                   jax.ShapeDtypeStruct((GRID, 8, 2 * C), jnp.float32)],
        compiler_params=pltpu.CompilerParams(
            dimension_semantics=("parallel",)),
    )(g, cpt, bias, scale, shift, wc, wg, b)


def _l3_body(hc_ref, hg_ref, sc_ref, sh_ref, wc_ref, wg_ref, b_ref,
             mx_ref, mn_ref, st_ref):
    xc = jnp.maximum(
        hc_ref[...].astype(jnp.float32) * sc_ref[:, :C] + sh_ref[:, :C], 0.0)
    xg = jnp.maximum(
        hg_ref[...].astype(jnp.float32) * sc_ref[:, C:] + sh_ref[:, C:], 0.0)
    yc = jnp.dot(xc.astype(jnp.bfloat16), wc_ref[...],
                 preferred_element_type=jnp.float32) + b_ref[:, :2 * C]
    yg = jnp.dot(xg.astype(jnp.bfloat16), wg_ref[...],
                 preferred_element_type=jnp.float32) + b_ref[:, 2 * C:]
    yc3 = yc.reshape(SG, K, 2 * C)
    yg3 = yg.reshape(SG, K, 2 * C)
    mx_ref[...] = jnp.concatenate([jnp.max(yc3, axis=1),
                                   jnp.max(yg3, axis=1)], axis=1)
    mn_ref[...] = jnp.concatenate([jnp.min(yc3, axis=1),
                                   jnp.min(yg3, axis=1)], axis=1)
    s = jnp.concatenate([jnp.sum(yc, axis=0, keepdims=True),
                         jnp.sum(yg, axis=0, keepdims=True)], axis=1)
    q = jnp.concatenate([jnp.sum(yc * yc, axis=0, keepdims=True),
                         jnp.sum(yg * yg, axis=0, keepdims=True)], axis=1)
    st_ref[...] = jnp.concatenate(
        [s, q, jnp.zeros((6, 4 * C), jnp.float32)], axis=0)[None]


def _l3(hc, hg, scale, shift, wc, wg, b):
    return pl.pallas_call(
        _l3_body,
        grid=(GRID,),
        in_specs=[pl.BlockSpec((TILE, C), lambda i: (i, 0)),
                  pl.BlockSpec((TILE, C), lambda i: (i, 0)),
                  pl.BlockSpec((1, 2 * C), lambda i: (0, 0)),
                  pl.BlockSpec((1, 2 * C), lambda i: (0, 0)),
                  pl.BlockSpec((C, 2 * C), lambda i: (0, 0)),
                  pl.BlockSpec((C, 2 * C), lambda i: (0, 0)),
                  pl.BlockSpec((1, 4 * C), lambda i: (0, 0))],
        out_specs=[pl.BlockSpec((SG, 4 * C), lambda i: (i, 0)),
                   pl.BlockSpec((SG, 4 * C), lambda i: (i, 0)),
                   pl.BlockSpec((1, 8, 4 * C), lambda i: (i, 0, 0))],
        out_shape=[jax.ShapeDtypeStruct((B * S, 4 * C), jnp.float32),
                   jax.ShapeDtypeStruct((B * S, 4 * C), jnp.float32),
                   jax.ShapeDtypeStruct((GRID, 8, 4 * C), jnp.float32)],
        compiler_params=pltpu.CompilerParams(
            dimension_semantics=("parallel",)),
    )(hc, hg, scale, shift, wc, wg, b)


def _fin_body(mx_ref, mn_ref, a_ref, c_ref, o_ref):
    a = a_ref[...]
    sel = jnp.where(a >= 0.0, mx_ref[...], mn_ref[...])
    v = jnp.maximum(sel * a + c_ref[...], 0.0)
    o_ref[0] = v[:, :2 * C]
    o_ref[1] = v[:, 2 * C:]


def _fin(mx, mn, a, c):
    return pl.pallas_call(
        _fin_body,
        grid=(B * S // TILE5,),
        in_specs=[pl.BlockSpec((TILE5, 4 * C), lambda i: (i, 0)),
                  pl.BlockSpec((TILE5, 4 * C), lambda i: (i, 0)),
                  pl.BlockSpec((1, 4 * C), lambda i: (0, 0)),
                  pl.BlockSpec((1, 4 * C), lambda i: (0, 0))],
        out_specs=pl.BlockSpec((2, TILE5, 2 * C), lambda i: (0, i, 0)),
        out_shape=jax.ShapeDtypeStruct((2, B * S, 2 * C), jnp.float32),
        compiler_params=pltpu.CompilerParams(
            dimension_semantics=("parallel",)),
    )(mx, mn, a, c)


def kernel(pos, feat, centroids, neighbor_idx, params):
    conv = params["conv"]
    geo = params["conv_geo"]

    w1 = conv[0]["W"]
    w1r_t = jnp.transpose(w1[:, :3])
    w1f_t = jnp.transpose(w1[:, 3:])
    g1 = geo[0]["W"]
    gsrc_t = jnp.transpose(g1[:, :3])
    gdst_t = jnp.transpose(g1[:, 3:])

    m16 = jnp.zeros((16, 2 * C), jnp.float32)
    m16 = m16.at[0:3, 0:C].set(w1r_t).at[0:3, C:].set(gsrc_t)
    mc16 = jnp.zeros((16, 2 * C), jnp.float32)
    mc16 = mc16.at[0:3, 0:C].set(-w1r_t).at[0:3, C:].set(gdst_t)
    bias1 = jnp.concatenate([conv[0]["b"], geo[0]["b"]]).reshape(1, 2 * C)

    w2c = conv[1]["W"].T.astype(jnp.bfloat16)
    w2g = geo[1]["W"].T.astype(jnp.bfloat16)
    bias2 = jnp.concatenate([conv[1]["b"], geo[1]["b"]]).reshape(1, 2 * C)

    w3c = conv[2]["W"].T.astype(jnp.bfloat16)
    w3g = geo[2]["W"].T.astype(jnp.bfloat16)
    bias3 = jnp.concatenate([conv[2]["b"], geo[2]["b"]]).reshape(1, 4 * C)

    gam1 = jnp.concatenate([conv[0]["gamma"], geo[0]["gamma"]])
    bet1 = jnp.concatenate([conv[0]["beta"], geo[0]["beta"]])
    gam2 = jnp.concatenate([conv[1]["gamma"], geo[1]["gamma"]])
    bet2 = jnp.concatenate([conv[1]["beta"], geo[1]["beta"]])
    gam3 = jnp.concatenate([conv[2]["gamma"], geo[2]["gamma"]])
    bet3 = jnp.concatenate([conv[2]["beta"], geo[2]["beta"]])

    feat2d = feat.reshape(B * N, C)
    p16 = jnp.zeros((B * N, 16), jnp.float32).at[:, :3].set(pos.reshape(-1, 3))
    nidx = neighbor_idx.reshape(-1).astype(jnp.int32)
    cent = centroids.reshape(-1).astype(jnp.int32)

    t, ct = _tables(feat2d, p16, w1f_t.astype(jnp.bfloat16), m16, mc16)
    g, cpt = _sc_gather(t, ct, nidx, cent)
    st1 = jnp.sum(_l1(g, cpt, bias1), axis=0)
    m = float(R)
    sc1, sh1 = _bn_affine(st1, gam1, bet1, m)
    h2c, h2g, st2p = _mid(g, cpt, bias1, sc1, sh1, w2c, w2g, bias2)
    sc2, sh2 = _bn_affine(jnp.sum(st2p, axis=0), gam2, bet2, m)
    mx, mn, st3p = _l3(h2c, h2g, sc2, sh2, w3c, w3g, bias3)
    st3 = jnp.sum(st3p, axis=0)
    sc3, sh3 = _bn_affine(st3, gam3, bet3, m)
    out = _fin(mx, mn, sc3, sh3)

    return out.reshape(2 * B * S, 2 * C)


# speedup vs baseline: 1.1242x; 1.0262x over previous
import functools

import jax
import jax.numpy as jnp
from jax import lax
from jax.experimental import pallas as pl
from jax.experimental.pallas import tpu as pltpu
from jax.experimental.pallas import tpu_sc as plsc

B, N, S, K = 8, 4096, 1024, 64
C = 128
EPS = 1e-5
R = B * S * K
TILE = 16384
TILE5 = 4096
GRID = R // TILE
SG = TILE // K
NW = 32
RPW = R // NW
CH = 128
CPW = (B * S) // NW

_HI = jax.lax.Precision.HIGHEST


def _bn_affine(stats, gamma, beta, m):
    s, q = stats[0], stats[1]
    mean = s / m
    var = q / m - mean * mean
    scale = gamma / jnp.sqrt(var + EPS)
    shift = beta - mean * scale
    return scale.reshape(1, -1), shift.reshape(1, -1)


def _pack(conv_f32, geo_f32):
    lo = lax.bitcast_convert_type(conv_f32.astype(jnp.bfloat16),
                                  jnp.uint16).astype(jnp.uint32)
    hi = lax.bitcast_convert_type(geo_f32.astype(jnp.bfloat16),
                                  jnp.uint16).astype(jnp.uint32)
    return lax.bitcast_convert_type(lo | (hi << 16), jnp.int32)


def _unpack(words_i32):
    w = lax.bitcast_convert_type(words_i32, jnp.uint32)
    conv = lax.bitcast_convert_type(w << 16, jnp.float32)
    geo = lax.bitcast_convert_type(w & jnp.uint32(0xFFFF0000), jnp.float32)
    return conv, geo


def _tables_body(f_ref, p_ref, wf_ref, m16_ref, mc16_ref, t_ref, ct_ref):
    t = jnp.dot(p_ref[...], m16_ref[...],
                preferred_element_type=jnp.float32, precision=_HI)
    tl = t[:, :C] + jnp.dot(f_ref[...].astype(jnp.bfloat16), wf_ref[...],
                            preferred_element_type=jnp.float32)
    t_ref[...] = _pack(tl, t[:, C:])
    ct = jnp.dot(p_ref[...], mc16_ref[...],
                 preferred_element_type=jnp.float32, precision=_HI)
    ct_ref[...] = _pack(ct[:, :C], ct[:, C:])


def _tables(feat2d, p16, w1f_t, m16, mc16):
    return pl.pallas_call(
        _tables_body,
        grid=(B * N // 512,),
        in_specs=[pl.BlockSpec((512, C), lambda i: (i, 0)),
                  pl.BlockSpec((512, 16), lambda i: (i, 0)),
                  pl.BlockSpec((C, C), lambda i: (0, 0)),
                  pl.BlockSpec((16, 2 * C), lambda i: (0, 0)),
                  pl.BlockSpec((16, 2 * C), lambda i: (0, 0))],
        out_specs=[pl.BlockSpec((512, C), lambda i: (i, 0)),
                   pl.BlockSpec((512, C), lambda i: (i, 0))],
        out_shape=[jax.ShapeDtypeStruct((B * N, C), jnp.int32),
                   jax.ShapeDtypeStruct((B * N, C), jnp.int32)],
        compiler_params=pltpu.CompilerParams(
            dimension_semantics=("parallel",)),
    )(feat2d, p16, w1f_t, m16, mc16)


def _sc_gather(t, ct, nidx, cent):
    mesh = plsc.VectorSubcoreMesh(core_axis_name="c", subcore_axis_name="s")

    @functools.partial(
        pl.kernel, mesh=mesh,
        out_type=[jax.ShapeDtypeStruct((R, C), jnp.int32),
                  jax.ShapeDtypeStruct((B * S, C), jnp.int32)],
        scratch_types=[pltpu.VMEM((CH,), jnp.int32),
                       pltpu.VMEM((CH, C), jnp.int32),
                       pltpu.SemaphoreType.DMA],
    )
    def k(t_hbm, ct_hbm, nidx_hbm, cent_hbm, g_hbm, cpt_hbm,
          idx_v, rows_v, sem):
        wid = lax.axis_index("s") * 2 + lax.axis_index("c")
        b_off = (wid // 4) * N

        for cb in range(CPW // CH):
            cbase = wid * CPW + cb * CH
            pltpu.sync_copy(cent_hbm.at[pl.ds(cbase, CH)], idx_v)
            for j in range(CH // 16):
                idx_v[pl.ds(j * 16, 16)] = idx_v[pl.ds(j * 16, 16)] + b_off
            pltpu.async_copy(ct_hbm.at[idx_v], rows_v, sem).wait()
            pltpu.sync_copy(rows_v, cpt_hbm.at[pl.ds(cbase, CH)])

        def body(i, carry):
            base = wid * RPW + i * CH
            pltpu.sync_copy(nidx_hbm.at[pl.ds(base, CH)], idx_v)
            for j in range(CH // 16):
                idx_v[pl.ds(j * 16, 16)] = idx_v[pl.ds(j * 16, 16)] + b_off
            pltpu.async_copy(t_hbm.at[idx_v], rows_v, sem).wait()
            pltpu.sync_copy(rows_v, g_hbm.at[pl.ds(base, CH)])
            return carry
        lax.fori_loop(0, RPW // CH, body, 0)

    return k(t, ct, nidx, cent)


def _l1_body(g_ref, cpt_ref, bias_ref, st_ref):
    cpc, cpg = _unpack(cpt_ref[...])
    gc, gg = _unpack(g_ref[...])
    ccc = cpc + bias_ref[:, :C]
    ccg = cpg + bias_ref[:, C:]
    hc = (gc.reshape(SG, K, C) + ccc[:, None, :]).reshape(TILE, C)
    hg = (gg.reshape(SG, K, C) + ccg[:, None, :]).reshape(TILE, C)
    s = jnp.concatenate([jnp.sum(hc, axis=0, keepdims=True),
                         jnp.sum(hg, axis=0, keepdims=True)], axis=1)
    q = jnp.concatenate([jnp.sum(hc * hc, axis=0, keepdims=True),
                         jnp.sum(hg * hg, axis=0, keepdims=True)], axis=1)
    st_ref[...] = jnp.concatenate(
        [s, q, jnp.zeros((6, 2 * C), jnp.float32)], axis=0)[None]


def _l1(g, cpt, bias):
    return pl.pallas_call(
        _l1_body,
        grid=(GRID,),
        in_specs=[pl.BlockSpec((TILE, C), lambda i: (i, 0)),
                  pl.BlockSpec((SG, C), lambda i: (i, 0)),
                  pl.BlockSpec((1, 2 * C), lambda i: (0, 0))],
        out_specs=pl.BlockSpec((1, 8, 2 * C), lambda i: (i, 0, 0)),
        out_shape=jax.ShapeDtypeStruct((GRID, 8, 2 * C), jnp.float32),
        compiler_params=pltpu.CompilerParams(
            dimension_semantics=("parallel",)),
    )(g, cpt, bias)


def _assemble_x1(g_ref, cpt_ref, bias_ref, sc_ref, sh_ref):
    cpc, cpg = _unpack(cpt_ref[...])
    gc, gg = _unpack(g_ref[...])
    ccc = cpc + bias_ref[:, :C]
    ccg = cpg + bias_ref[:, C:]
    hc = (gc.reshape(SG, K, C) + ccc[:, None, :]).reshape(TILE, C)
    hg = (gg.reshape(SG, K, C) + ccg[:, None, :]).reshape(TILE, C)
    xc = jnp.maximum(hc * sc_ref[:, :C] + sh_ref[:, :C], 0.0)
    xg = jnp.maximum(hg * sc_ref[:, C:] + sh_ref[:, C:], 0.0)
    return xc, xg


def _mid_body(g_ref, cpt_ref, bias_ref, sc_ref, sh_ref, wc_ref, wg_ref,
              b_ref, oc_ref, og_ref, st_ref):
    xc, xg = _assemble_x1(g_ref, cpt_ref, bias_ref, sc_ref, sh_ref)
    yc = jnp.dot(xc.astype(jnp.bfloat16), wc_ref[...],
                 preferred_element_type=jnp.float32) + b_ref[:, :C]
    yg = jnp.dot(xg.astype(jnp.bfloat16), wg_ref[...],
                 preferred_element_type=jnp.float32) + b_ref[:, C:]
    oc_ref[...] = yc.astype(jnp.bfloat16)
    og_ref[...] = yg.astype(jnp.bfloat16)
    s = jnp.concatenate([jnp.sum(yc, axis=0, keepdims=True),
                         jnp.sum(yg, axis=0, keepdims=True)], axis=1)
    q = jnp.concatenate([jnp.sum(yc * yc, axis=0, keepdims=True),
                         jnp.sum(yg * yg, axis=0, keepdims=True)], axis=1)
    st_ref[...] = jnp.concatenate(
        [s, q, jnp.zeros((6, 2 * C), jnp.float32)], axis=0)[None]


def _mid(g, cpt, bias, scale, shift, wc, wg, b):
    return pl.pallas_call(
        _mid_body,
        grid=(GRID,),
        in_specs=[pl.BlockSpec((TILE, C), lambda i: (i, 0)),
                  pl.BlockSpec((SG, C), lambda i: (i, 0)),
                  pl.BlockSpec((1, 2 * C), lambda i: (0, 0)),
                  pl.BlockSpec((1, 2 * C), lambda i: (0, 0)),
                  pl.BlockSpec((1, 2 * C), lambda i: (0, 0)),
                  pl.BlockSpec((C, C), lambda i: (0, 0)),
                  pl.BlockSpec((C, C), lambda i: (0, 0)),
                  pl.BlockSpec((1, 2 * C), lambda i: (0, 0))],
        out_specs=[pl.BlockSpec((TILE, C), lambda i: (i, 0)),
                   pl.BlockSpec((TILE, C), lambda i: (i, 0)),
                   pl.BlockSpec((1, 8, 2 * C), lambda i: (i, 0, 0))],
        out_shape=[jax.ShapeDtypeStruct((R, C), jnp.bfloat16),
                   jax.ShapeDtypeStruct((R, C), jnp.bfloat16),
                   jax.ShapeDtypeStruct((GRID, 8, 2 * C), jnp.float32)],
        compiler_params=pltpu.CompilerParams(
            dimension_semantics=("parallel",)),
    )(g, cpt, bias, scale, shift, wc, wg, b)


def _l3_body(hc_ref, hg_ref, sc_ref, sh_ref, wc_ref, wg_ref, b_ref,
             mx_ref, mn_ref, st_ref):
    xc = jnp.maximum(
        hc_ref[...].astype(jnp.float32) * sc_ref[:, :C] + sh_ref[:, :C], 0.0)
    xg = jnp.maximum(
        hg_ref[...].astype(jnp.float32) * sc_ref[:, C:] + sh_ref[:, C:], 0.0)
    yc = jnp.dot(xc.astype(jnp.bfloat16), wc_ref[...],
                 preferred_element_type=jnp.float32) + b_ref[:, :2 * C]
    yg = jnp.dot(xg.astype(jnp.bfloat16), wg_ref[...],
                 preferred_element_type=jnp.float32) + b_ref[:, 2 * C:]
    yc3 = yc.reshape(SG, K, 2 * C)
    yg3 = yg.reshape(SG, K, 2 * C)
    mx_ref[...] = jnp.concatenate([jnp.max(yc3, axis=1),
                                   jnp.max(yg3, axis=1)], axis=1)
    mn_ref[...] = jnp.concatenate([jnp.min(yc3, axis=1),
                                   jnp.min(yg3, axis=1)], axis=1)
    s = jnp.concatenate([jnp.sum(yc, axis=0, keepdims=True),
                         jnp.sum(yg, axis=0, keepdims=True)], axis=1)
    q = jnp.concatenate([jnp.sum(yc * yc, axis=0, keepdims=True),
                         jnp.sum(yg * yg, axis=0, keepdims=True)], axis=1)
    st_ref[...] = jnp.concatenate(
        [s, q, jnp.zeros((6, 4 * C), jnp.float32)], axis=0)[None]


def _l3(hc, hg, scale, shift, wc, wg, b):
    return pl.pallas_call(
        _l3_body,
        grid=(GRID,),
        in_specs=[pl.BlockSpec((TILE, C), lambda i: (i, 0)),
                  pl.BlockSpec((TILE, C), lambda i: (i, 0)),
                  pl.BlockSpec((1, 2 * C), lambda i: (0, 0)),
                  pl.BlockSpec((1, 2 * C), lambda i: (0, 0)),
                  pl.BlockSpec((C, 2 * C), lambda i: (0, 0)),
                  pl.BlockSpec((C, 2 * C), lambda i: (0, 0)),
                  pl.BlockSpec((1, 4 * C), lambda i: (0, 0))],
        out_specs=[pl.BlockSpec((SG, 4 * C), lambda i: (i, 0)),
                   pl.BlockSpec((SG, 4 * C), lambda i: (i, 0)),
                   pl.BlockSpec((1, 8, 4 * C), lambda i: (i, 0, 0))],
        out_shape=[jax.ShapeDtypeStruct((B * S, 4 * C), jnp.float32),
                   jax.ShapeDtypeStruct((B * S, 4 * C), jnp.float32),
                   jax.ShapeDtypeStruct((GRID, 8, 4 * C), jnp.float32)],
        compiler_params=pltpu.CompilerParams(
            dimension_semantics=("parallel",)),
    )(hc, hg, scale, shift, wc, wg, b)


def _fin_body(mx_ref, mn_ref, a_ref, c_ref, o_ref):
    a = a_ref[...]
    sel = jnp.where(a >= 0.0, mx_ref[...], mn_ref[...])
    v = jnp.maximum(sel * a + c_ref[...], 0.0)
    o_ref[0] = v[:, :2 * C]
    o_ref[1] = v[:, 2 * C:]


def _fin(mx, mn, a, c):
    return pl.pallas_call(
        _fin_body,
        grid=(B * S // TILE5,),
        in_specs=[pl.BlockSpec((TILE5, 4 * C), lambda i: (i, 0)),
                  pl.BlockSpec((TILE5, 4 * C), lambda i: (i, 0)),
                  pl.BlockSpec((1, 4 * C), lambda i: (0, 0)),
                  pl.BlockSpec((1, 4 * C), lambda i: (0, 0))],
        out_specs=pl.BlockSpec((2, TILE5, 2 * C), lambda i: (0, i, 0)),
        out_shape=jax.ShapeDtypeStruct((2, B * S, 2 * C), jnp.float32),
        compiler_params=pltpu.CompilerParams(
            dimension_semantics=("parallel",)),
    )(mx, mn, a, c)


def kernel(pos, feat, centroids, neighbor_idx, params):
    conv = params["conv"]
    geo = params["conv_geo"]

    w1 = conv[0]["W"]
    w1r_t = jnp.transpose(w1[:, :3])
    w1f_t = jnp.transpose(w1[:, 3:])
    g1 = geo[0]["W"]
    gsrc_t = jnp.transpose(g1[:, :3])
    gdst_t = jnp.transpose(g1[:, 3:])

    m16 = jnp.zeros((16, 2 * C), jnp.float32)
    m16 = m16.at[0:3, 0:C].set(w1r_t).at[0:3, C:].set(gsrc_t)
    mc16 = jnp.zeros((16, 2 * C), jnp.float32)
    mc16 = mc16.at[0:3, 0:C].set(-w1r_t).at[0:3, C:].set(gdst_t)
    bias1 = jnp.concatenate([conv[0]["b"], geo[0]["b"]]).reshape(1, 2 * C)

    w2c = conv[1]["W"].T.astype(jnp.bfloat16)
    w2g = geo[1]["W"].T.astype(jnp.bfloat16)
    bias2 = jnp.concatenate([conv[1]["b"], geo[1]["b"]]).reshape(1, 2 * C)

    w3c = conv[2]["W"].T.astype(jnp.bfloat16)
    w3g = geo[2]["W"].T.astype(jnp.bfloat16)
    bias3 = jnp.concatenate([conv[2]["b"], geo[2]["b"]]).reshape(1, 4 * C)

    gam1 = jnp.concatenate([conv[0]["gamma"], geo[0]["gamma"]])
    bet1 = jnp.concatenate([conv[0]["beta"], geo[0]["beta"]])
    gam2 = jnp.concatenate([conv[1]["gamma"], geo[1]["gamma"]])
    bet2 = jnp.concatenate([conv[1]["beta"], geo[1]["beta"]])
    gam3 = jnp.concatenate([conv[2]["gamma"], geo[2]["gamma"]])
    bet3 = jnp.concatenate([conv[2]["beta"], geo[2]["beta"]])

    feat2d = feat.reshape(B * N, C)
    p16 = jnp.zeros((B * N, 16), jnp.float32).at[:, :3].set(pos.reshape(-1, 3))
    nidx = neighbor_idx.reshape(-1).astype(jnp.int32)
    cent = centroids.reshape(-1).astype(jnp.int32)

    t, ct = _tables(feat2d, p16, w1f_t.astype(jnp.bfloat16), m16, mc16)
    g, cpt = _sc_gather(t, ct, nidx, cent)
    st1 = jnp.sum(_l1(g, cpt, bias1), axis=0)
    m = float(R)
    sc1, sh1 = _bn_affine(st1, gam1, bet1, m)
    h2c, h2g, st2p = _mid(g, cpt, bias1, sc1, sh1, w2c, w2g, bias2)
    sc2, sh2 = _bn_affine(jnp.sum(st2p, axis=0), gam2, bet2, m)
    mx, mn, st3p = _l3(h2c, h2g, sc2, sh2, w3c, w3g, bias3)
    st3 = jnp.sum(st3p, axis=0)
    sc3, sh3 = _bn_affine(st3, gam3, bet3, m)
    out = _fin(mx, mn, sc3, sh3)

    return out.reshape(2 * B * S, 2 * C)
